# baseline (speedup 1.0000x reference)
import functools

import jax
import jax.numpy as jnp
import numpy as np
from jax import lax
from jax.experimental import pallas as pl
from jax.experimental.pallas import tpu as pltpu
from jax.experimental.pallas import tpu_sc as plsc

N = 10000
D = 128
E = 320000
T = 2
P = 768
B = 256
NH = 4
HD = D // NH

NC = 2
NS = 16
NW = NC * NS
CS = 128
NCH_H = 104
NCH_L = 56
HEAVY_CID = 1
E_PAD = NS * CS * (NCH_H + NCH_L)
NP = 10240
RPS = NP // NS

_mesh_cache = []


def _mesh():
  if not _mesh_cache:
    _mesh_cache.append(
        plsc.VectorSubcoreMesh(core_axis_name="c", subcore_axis_name="s",
                               num_cores=NC, num_subcores=NS))
  return _mesh_cache[0]


def _segmean_partials(x, src_idx, dst_idx, want_deg=True):
  @functools.partial(
      pl.kernel, mesh=_mesh(),
      out_type=[jax.ShapeDtypeStruct((NC, NP, D), jnp.float32),
                jax.ShapeDtypeStruct((NC, NP), jnp.float32)],
      scratch_types=[
          pltpu.VMEM((NCH_H, CS), jnp.int32),
          pltpu.VMEM((NCH_H, CS), jnp.int32),
          pltpu.VMEM((CS, D), jnp.float32),
          pltpu.VMEM((CS,), jnp.float32),
          pltpu.VMEM_SHARED((NP, D), jnp.float32),
          pltpu.VMEM_SHARED((NP,), jnp.float32),
          pltpu.SemaphoreType.DMA,
      ],
  )
  def k(x_hbm, src_hbm, dst_hbm, part_hbm, deg_hbm,
        src_v, dst_v, r0, ones_v, acc_sh, dacc_sh, s0):
    cid = lax.axis_index("c")
    sid = lax.axis_index("s")
    wid = sid * NC + cid

    zeros16 = jnp.zeros((16,), jnp.float32)
    ones16 = jnp.ones((16,), jnp.float32)

    @pl.loop(0, CS)
    def _(r):
      for c in range(D // 16):
        r0[r, pl.ds(c * 16, 16)] = zeros16
    if want_deg:
      for c in range(CS // 16):
        ones_v[pl.ds(c * 16, 16)] = ones16

    for t in range(RPS // CS):
      pltpu.sync_copy(r0, acc_sh.at[pl.ds(sid * RPS + t * CS, CS)])
    if want_deg:
      for t in range(RPS // D):
        pltpu.sync_copy(r0.at[0],
                        dacc_sh.at[pl.ds(sid * RPS + t * D, D)])
    plsc.subcore_barrier()

    pltpu.sync_copy(src_hbm.at[wid], src_v)
    pltpu.sync_copy(dst_hbm.at[wid], dst_v)

    nch = jnp.where(cid == HEAVY_CID, NCH_H, NCH_L)

    @pl.loop(0, nch)
    def _(j):
      pltpu.async_copy(x_hbm.at[src_v.at[j]], r0, s0).wait()
      pltpu.sync_copy(r0, acc_sh.at[dst_v.at[j]], add=True)
      if want_deg:
        pltpu.sync_copy(ones_v, dacc_sh.at[dst_v.at[j]], add=True)

    plsc.subcore_barrier()

    pltpu.sync_copy(acc_sh.at[pl.ds(sid * RPS, RPS)],
                    part_hbm.at[cid, pl.ds(sid * RPS, RPS)])
    if want_deg:
      pltpu.sync_copy(dacc_sh.at[pl.ds(sid * RPS, RPS)],
                      deg_hbm.at[cid, pl.ds(sid * RPS, RPS)])

  return k(x, src_idx, dst_idx)


_PPW = P // NW


def _gather_rows(table, idx):
  @functools.partial(
      pl.kernel, mesh=_mesh(),
      out_type=jax.ShapeDtypeStruct((P, D), jnp.float32),
      scratch_types=[
          pltpu.VMEM((_PPW,), jnp.int32),
          pltpu.VMEM((_PPW, D), jnp.float32),
          pltpu.SemaphoreType.DMA,
      ],
  )
  def k(tab_hbm, idx_hbm, out_hbm, idx_v, rows_v, sem):
    wid = lax.axis_index("s") * NC + lax.axis_index("c")
    base = wid * _PPW
    pltpu.sync_copy(idx_hbm.at[pl.ds(base, _PPW)], idx_v)
    pltpu.async_copy(tab_hbm.at[idx_v], rows_v, sem).wait()
    pltpu.sync_copy(rows_v, out_hbm.at[pl.ds(base, _PPW)])

  return k(table, idx)


_BR = 2000


def _sage_linear(part, degT, x, Wl, bl, Wr):
  def body(part_ref, degT_ref, x_ref, Wl_ref, bl_ref, Wr_ref, out_ref):
    p = part_ref[0] + part_ref[1]
    deg = degT_ref[:, 0] + degT_ref[:, 1]
    mean = p / jnp.maximum(deg, 1.0)[:, None]
    h = (jnp.dot(mean, Wl_ref[...], preferred_element_type=jnp.float32)
         + bl_ref[...]
         + jnp.dot(x_ref[...], Wr_ref[...], preferred_element_type=jnp.float32))
    out_ref[...] = jnp.maximum(h, 0.0)

  return pl.pallas_call(
      body,
      grid=(N // _BR,),
      in_specs=[
          pl.BlockSpec((NC, _BR, D), lambda i: (0, i, 0)),
          pl.BlockSpec((_BR, NC), lambda i: (i, 0)),
          pl.BlockSpec((_BR, D), lambda i: (i, 0)),
          pl.BlockSpec((D, D), lambda i: (0, 0)),
          pl.BlockSpec((1, D), lambda i: (0, 0)),
          pl.BlockSpec((D, D), lambda i: (0, 0)),
      ],
      out_specs=pl.BlockSpec((_BR, D), lambda i: (i, 0)),
      out_shape=jax.ShapeDtypeStruct((N, D), jnp.float32),
  )(part, degT, x, Wl, bl, Wr)


def _layer2_meta(part20, part21, degT0, degT1, h0, h1,
                 Wl20, bl20, Wr20, Wl21, bl21, Wr21, Wm, bm, vm):
  def body(p0_ref, p1_ref, d0_ref, d1_ref, h0_ref, h1_ref,
           Wl20_ref, bl20_ref, Wr20_ref, Wl21_ref, bl21_ref, Wr21_ref,
           Wm_ref, bm_ref, vm_ref, out_ref):
    h2 = []
    for (p_ref, d_ref, h_ref, Wl_ref, bl_ref, Wr_ref) in (
        (p0_ref, d0_ref, h0_ref, Wl20_ref, bl20_ref, Wr20_ref),
        (p1_ref, d1_ref, h1_ref, Wl21_ref, bl21_ref, Wr21_ref)):
      p = p_ref[0] + p_ref[1]
      deg = d_ref[:, 0] + d_ref[:, 1]
      mean = p / jnp.maximum(deg, 1.0)[:, None]
      h = (jnp.dot(mean, Wl_ref[...], preferred_element_type=jnp.float32)
           + bl_ref[...]
           + jnp.dot(h_ref[...], Wr_ref[...], preferred_element_type=jnp.float32))
      h2.append(jnp.maximum(h, 0.0))
    Wm_ = Wm_ref[...]
    bm_ = bm_ref[...]
    vm_ = vm_ref[...]
    s0 = jnp.dot(jnp.tanh(jnp.dot(h2[0], Wm_,
                                  preferred_element_type=jnp.float32) + bm_),
                 vm_, preferred_element_type=jnp.float32)
    s1 = jnp.dot(jnp.tanh(jnp.dot(h2[1], Wm_,
                                  preferred_element_type=jnp.float32) + bm_),
                 vm_, preferred_element_type=jnp.float32)
    m = jnp.maximum(s0, s1)
    e0 = jnp.exp(s0 - m)
    e1 = jnp.exp(s1 - m)
    w0 = e0 / (e0 + e1)
    out_ref[...] = w0 * h2[0] + (1.0 - w0) * h2[1]

  wspec = pl.BlockSpec((D, D), lambda i: (0, 0))
  bspec = pl.BlockSpec((1, D), lambda i: (0, 0))
  return pl.pallas_call(
      body,
      grid=(N // _BR,),
      in_specs=[
          pl.BlockSpec((NC, _BR, D), lambda i: (0, i, 0)),
          pl.BlockSpec((NC, _BR, D), lambda i: (0, i, 0)),
          pl.BlockSpec((_BR, NC), lambda i: (i, 0)),
          pl.BlockSpec((_BR, NC), lambda i: (i, 0)),
          pl.BlockSpec((_BR, D), lambda i: (i, 0)),
          pl.BlockSpec((_BR, D), lambda i: (i, 0)),
          wspec, bspec, wspec, wspec, bspec, wspec,
          wspec, bspec, pl.BlockSpec((D, 1), lambda i: (0, 0)),
      ],
      out_specs=pl.BlockSpec((_BR, D), lambda i: (i, 0)),
      out_shape=jax.ShapeDtypeStruct((N, D), jnp.float32),
  )(part20, part21, degT0, degT1, h0, h1,
    Wl20, bl20, Wr20, Wl21, bl21, Wr21, Wm, bm, vm)


def _hypersagnn(emb, bcol, brow, Wst, bst, Wq, bq, Wk, bk, Wv, bv, Wo, bo,
                beta, Wp, bp):
  def body(emb_ref, bcol_ref, brow_ref, Wst_ref, bst_ref,
           Wq_ref, bq_ref, Wk_ref, bk_ref, Wv_ref, bv_ref, Wo_ref, bo_ref,
           beta_ref, Wp_ref, bp_ref, out_ref):
    x = emb_ref[...]
    static = jnp.maximum(
        jnp.dot(x, Wst_ref[...], preferred_element_type=jnp.float32)
        + bst_ref[...], 0.0)
    bc = bcol_ref[...]
    br = brow_ref[...]
    ri = lax.broadcasted_iota(jnp.int32, (P, P), 0)
    ci = lax.broadcasted_iota(jnp.int32, (P, P), 1)
    valid = (bc == br) & (ri != ci)
    scale = 1.0 / np.sqrt(float(HD))
    for l in range(2):
      Q = jnp.dot(x, Wq_ref[l], preferred_element_type=jnp.float32) + bq_ref[l]
      K = jnp.dot(x, Wk_ref[l], preferred_element_type=jnp.float32) + bk_ref[l]
      V = jnp.dot(x, Wv_ref[l], preferred_element_type=jnp.float32) + bv_ref[l]
      cols = []
      for h in range(NH):
        Qh = Q[:, h * HD:(h + 1) * HD]
        Kh = K[:, h * HD:(h + 1) * HD]
        Vh = V[:, h * HD:(h + 1) * HD]
        a = lax.dot_general(Qh, Kh, (((1,), (1,)), ((), ())),
                            preferred_element_type=jnp.float32) * scale
        a = jnp.where(valid, a, -1e9)
        m = jnp.max(a, axis=1, keepdims=True)
        ex = jnp.exp(a - m)
        w = ex / jnp.sum(ex, axis=1, keepdims=True)
        cols.append(jnp.dot(w, Vh, preferred_element_type=jnp.float32))
      ctx = jnp.concatenate(cols, axis=1)
      x = x + beta_ref[l, 0] * (
          jnp.dot(ctx, Wo_ref[l], preferred_element_type=jnp.float32)
          + bo_ref[l])
    diff = x - static
    node = jnp.dot(diff * diff, Wp_ref[...],
                   preferred_element_type=jnp.float32) + bp_ref[...]
    colid = lax.broadcasted_iota(jnp.int32, (P, B), 1).astype(jnp.float32)
    onehot = (bc == colid).astype(jnp.float32)
    ssum = lax.dot_general(onehot, node, (((0,), (0,)), ((), ())),
                           preferred_element_type=jnp.float32)
    ones_col = jnp.ones((P, 1), jnp.float32)
    cnt = lax.dot_general(onehot, ones_col, (((0,), (0,)), ((), ())),
                          preferred_element_type=jnp.float32)
    out_ref[...] = ssum / jnp.maximum(cnt, 1.0)

  full = lambda s: pl.BlockSpec(s, lambda: tuple(0 for _ in s))
  return pl.pallas_call(
      body,
      in_specs=[
          full((P, D)), full((P, 1)), full((1, P)),
          full((D, D)), full((1, D)),
          full((2, D, D)), full((2, 1, D)),
          full((2, D, D)), full((2, 1, D)),
          full((2, D, D)), full((2, 1, D)),
          full((2, D, D)), full((2, 1, D)),
          full((2, 1)), full((D, 1)), full((1, 1)),
      ],
      out_specs=full((B, 1)),
      out_shape=jax.ShapeDtypeStruct((B, 1), jnp.float32),
  )(emb, bcol, brow, Wst, bst, Wq, bq, Wk, bk, Wv, bv, Wo, bo, beta, Wp, bp)


def _arrange(v, fill):
  EH = NS * NCH_H * CS
  heavy = v[:EH].reshape(NS, NCH_H, CS)
  light = jnp.concatenate(
      [v[EH:].reshape(NS, NCH_L, CS),
       jnp.full((NS, NCH_H - NCH_L, CS), fill, jnp.int32)], axis=1)
  pair = (light, heavy) if HEAVY_CID == 1 else (heavy, light)
  return jnp.stack(pair, axis=1).reshape(NW, NCH_H, CS)


def _prep_edges(src, dst):
  pad = E_PAD - E
  src = jnp.concatenate([src.astype(jnp.int32), jnp.zeros((pad,), jnp.int32)])
  dst = jnp.concatenate([dst.astype(jnp.int32),
                         jnp.full((pad,), NP - 1, jnp.int32)])
  return _arrange(src, 0), _arrange(dst, NP - 1)


def kernel(edge_index, pert_indices, batch_indices, gene_emb,
           Wl1, bl1, Wr1, Wl2, bl2, Wr2, Wm, bm, vm, Wst, bst,
           Wq, bq, Wk, bk, Wv, bv, Wo, bo, beta, Wp, bp):
  src0, dst0 = _prep_edges(edge_index[0, 0], edge_index[0, 1])
  src1, dst1 = _prep_edges(edge_index[1, 0], edge_index[1, 1])

  part10, deg0 = _segmean_partials(gene_emb, src0, dst0)
  part11, deg1 = _segmean_partials(gene_emb, src1, dst1)
  degT0 = deg0.T
  degT1 = deg1.T

  h10 = _sage_linear(part10, degT0, gene_emb,
                     Wl1[0], bl1[0].reshape(1, D), Wr1[0])
  h11 = _sage_linear(part11, degT1, gene_emb,
                     Wl1[1], bl1[1].reshape(1, D), Wr1[1])

  part20, _ = _segmean_partials(h10, src0, dst0, want_deg=False)
  part21, _ = _segmean_partials(h11, src1, dst1, want_deg=False)

  meta = _layer2_meta(part20, part21, degT0, degT1, h10, h11,
                      Wl2[0], bl2[0].reshape(1, D), Wr2[0],
                      Wl2[1], bl2[1].reshape(1, D), Wr2[1],
                      Wm, bm.reshape(1, D), vm.reshape(D, 1))

  emb = _gather_rows(meta, pert_indices.astype(jnp.int32))

  bf = batch_indices.astype(jnp.float32)
  scores = _hypersagnn(emb, bf.reshape(P, 1), bf.reshape(1, P),
                       Wst, bst.reshape(1, D),
                       Wq, bq.reshape(2, 1, D), Wk, bk.reshape(2, 1, D),
                       Wv, bv.reshape(2, 1, D), Wo, bo.reshape(2, 1, D),
                       beta.reshape(2, 1), Wp, bp.reshape(1, 1))
  return scores[:, 0]


# speedup vs baseline: 1.0641x; 1.0641x over previous
import functools

import jax
import jax.numpy as jnp
import numpy as np
from jax import lax
from jax.experimental import pallas as pl
from jax.experimental.pallas import tpu as pltpu
from jax.experimental.pallas import tpu_sc as plsc

N = 10000
D = 128
E = 320000
T = 2
P = 768
B = 256
NH = 4
HD = D // NH

NC = 2
NS = 16
NW = NC * NS
CS = 128
NCH_H = 104
NCH_L = 56
HEAVY_CID = 0
E_PAD = NS * CS * (NCH_H + NCH_L)
NP = 10240
RPS = NP // NS

_mesh_cache = []


def _mesh():
  if not _mesh_cache:
    _mesh_cache.append(
        plsc.VectorSubcoreMesh(core_axis_name="c", subcore_axis_name="s",
                               num_cores=NC, num_subcores=NS))
  return _mesh_cache[0]


def _segmean_partials(x, src_idx, dst_idx, want_deg=True):
  @functools.partial(
      pl.kernel, mesh=_mesh(),
      out_type=[jax.ShapeDtypeStruct((NC, NP, D), jnp.float32),
                jax.ShapeDtypeStruct((NC, NP), jnp.float32)],
      scratch_types=[
          pltpu.VMEM((NCH_H, CS), jnp.int32),
          pltpu.VMEM((NCH_H, CS), jnp.int32),
          pltpu.VMEM((CS, D), jnp.float32),
          pltpu.VMEM((CS,), jnp.float32),
          pltpu.VMEM_SHARED((NP, D), jnp.float32),
          pltpu.VMEM_SHARED((NP,), jnp.float32),
          pltpu.SemaphoreType.DMA,
      ],
  )
  def k(x_hbm, src_hbm, dst_hbm, part_hbm, deg_hbm,
        src_v, dst_v, r0, ones_v, acc_sh, dacc_sh, s0):
    cid = lax.axis_index("c")
    sid = lax.axis_index("s")
    wid = sid * NC + cid

    zeros16 = jnp.zeros((16,), jnp.float32)
    ones16 = jnp.ones((16,), jnp.float32)

    @pl.loop(0, CS)
    def _(r):
      for c in range(D // 16):
        r0[r, pl.ds(c * 16, 16)] = zeros16
    if want_deg:
      for c in range(CS // 16):
        ones_v[pl.ds(c * 16, 16)] = ones16

    for t in range(RPS // CS):
      pltpu.sync_copy(r0, acc_sh.at[pl.ds(sid * RPS + t * CS, CS)])
    if want_deg:
      for t in range(RPS // D):
        pltpu.sync_copy(r0.at[0],
                        dacc_sh.at[pl.ds(sid * RPS + t * D, D)])
    plsc.subcore_barrier()

    pltpu.sync_copy(src_hbm.at[wid], src_v)
    pltpu.sync_copy(dst_hbm.at[wid], dst_v)

    nch = jnp.where(cid == HEAVY_CID, NCH_H, NCH_L)

    @pl.loop(0, nch)
    def _(j):
      pltpu.async_copy(x_hbm.at[src_v.at[j]], r0, s0).wait()
      pltpu.sync_copy(r0, acc_sh.at[dst_v.at[j]], add=True)
      if want_deg:
        pltpu.sync_copy(ones_v, dacc_sh.at[dst_v.at[j]], add=True)

    plsc.subcore_barrier()

    pltpu.sync_copy(acc_sh.at[pl.ds(sid * RPS, RPS)],
                    part_hbm.at[cid, pl.ds(sid * RPS, RPS)])
    if want_deg:
      pltpu.sync_copy(dacc_sh.at[pl.ds(sid * RPS, RPS)],
                      deg_hbm.at[cid, pl.ds(sid * RPS, RPS)])

  return k(x, src_idx, dst_idx)


_PPW = P // NW


def _gather_rows(table, idx):
  @functools.partial(
      pl.kernel, mesh=_mesh(),
      out_type=jax.ShapeDtypeStruct((P, D), jnp.float32),
      scratch_types=[
          pltpu.VMEM((_PPW,), jnp.int32),
          pltpu.VMEM((_PPW, D), jnp.float32),
          pltpu.SemaphoreType.DMA,
      ],
  )
  def k(tab_hbm, idx_hbm, out_hbm, idx_v, rows_v, sem):
    wid = lax.axis_index("s") * NC + lax.axis_index("c")
    base = wid * _PPW
    pltpu.sync_copy(idx_hbm.at[pl.ds(base, _PPW)], idx_v)
    pltpu.async_copy(tab_hbm.at[idx_v], rows_v, sem).wait()
    pltpu.sync_copy(rows_v, out_hbm.at[pl.ds(base, _PPW)])

  return k(table, idx)


_BR = 2000


def _sage_linear(part, degT, x, Wl, bl, Wr):
  def body(part_ref, degT_ref, x_ref, Wl_ref, bl_ref, Wr_ref, out_ref):
    p = part_ref[0] + part_ref[1]
    deg = degT_ref[:, 0] + degT_ref[:, 1]
    mean = p / jnp.maximum(deg, 1.0)[:, None]
    h = (jnp.dot(mean, Wl_ref[...], preferred_element_type=jnp.float32)
         + bl_ref[...]
         + jnp.dot(x_ref[...], Wr_ref[...], preferred_element_type=jnp.float32))
    out_ref[...] = jnp.maximum(h, 0.0)

  return pl.pallas_call(
      body,
      grid=(N // _BR,),
      in_specs=[
          pl.BlockSpec((NC, _BR, D), lambda i: (0, i, 0)),
          pl.BlockSpec((_BR, NC), lambda i: (i, 0)),
          pl.BlockSpec((_BR, D), lambda i: (i, 0)),
          pl.BlockSpec((D, D), lambda i: (0, 0)),
          pl.BlockSpec((1, D), lambda i: (0, 0)),
          pl.BlockSpec((D, D), lambda i: (0, 0)),
      ],
      out_specs=pl.BlockSpec((_BR, D), lambda i: (i, 0)),
      out_shape=jax.ShapeDtypeStruct((N, D), jnp.float32),
  )(part, degT, x, Wl, bl, Wr)


def _layer2_meta(part20, part21, degT0, degT1, h0, h1,
                 Wl20, bl20, Wr20, Wl21, bl21, Wr21, Wm, bm, vm):
  def body(p0_ref, p1_ref, d0_ref, d1_ref, h0_ref, h1_ref,
           Wl20_ref, bl20_ref, Wr20_ref, Wl21_ref, bl21_ref, Wr21_ref,
           Wm_ref, bm_ref, vm_ref, out_ref):
    h2 = []
    for (p_ref, d_ref, h_ref, Wl_ref, bl_ref, Wr_ref) in (
        (p0_ref, d0_ref, h0_ref, Wl20_ref, bl20_ref, Wr20_ref),
        (p1_ref, d1_ref, h1_ref, Wl21_ref, bl21_ref, Wr21_ref)):
      p = p_ref[0] + p_ref[1]
      deg = d_ref[:, 0] + d_ref[:, 1]
      mean = p / jnp.maximum(deg, 1.0)[:, None]
      h = (jnp.dot(mean, Wl_ref[...], preferred_element_type=jnp.float32)
           + bl_ref[...]
           + jnp.dot(h_ref[...], Wr_ref[...], preferred_element_type=jnp.float32))
      h2.append(jnp.maximum(h, 0.0))
    Wm_ = Wm_ref[...]
    bm_ = bm_ref[...]
    vm_ = vm_ref[...]
    s0 = jnp.dot(jnp.tanh(jnp.dot(h2[0], Wm_,
                                  preferred_element_type=jnp.float32) + bm_),
                 vm_, preferred_element_type=jnp.float32)
    s1 = jnp.dot(jnp.tanh(jnp.dot(h2[1], Wm_,
                                  preferred_element_type=jnp.float32) + bm_),
                 vm_, preferred_element_type=jnp.float32)
    m = jnp.maximum(s0, s1)
    e0 = jnp.exp(s0 - m)
    e1 = jnp.exp(s1 - m)
    w0 = e0 / (e0 + e1)
    out_ref[...] = w0 * h2[0] + (1.0 - w0) * h2[1]

  wspec = pl.BlockSpec((D, D), lambda i: (0, 0))
  bspec = pl.BlockSpec((1, D), lambda i: (0, 0))
  return pl.pallas_call(
      body,
      grid=(N // _BR,),
      in_specs=[
          pl.BlockSpec((NC, _BR, D), lambda i: (0, i, 0)),
          pl.BlockSpec((NC, _BR, D), lambda i: (0, i, 0)),
          pl.BlockSpec((_BR, NC), lambda i: (i, 0)),
          pl.BlockSpec((_BR, NC), lambda i: (i, 0)),
          pl.BlockSpec((_BR, D), lambda i: (i, 0)),
          pl.BlockSpec((_BR, D), lambda i: (i, 0)),
          wspec, bspec, wspec, wspec, bspec, wspec,
          wspec, bspec, pl.BlockSpec((D, 1), lambda i: (0, 0)),
      ],
      out_specs=pl.BlockSpec((_BR, D), lambda i: (i, 0)),
      out_shape=jax.ShapeDtypeStruct((N, D), jnp.float32),
  )(part20, part21, degT0, degT1, h0, h1,
    Wl20, bl20, Wr20, Wl21, bl21, Wr21, Wm, bm, vm)


def _hypersagnn(emb, bcol, brow, Wst, bst, Wq, bq, Wk, bk, Wv, bv, Wo, bo,
                beta, Wp, bp):
  def body(emb_ref, bcol_ref, brow_ref, Wst_ref, bst_ref,
           Wq_ref, bq_ref, Wk_ref, bk_ref, Wv_ref, bv_ref, Wo_ref, bo_ref,
           beta_ref, Wp_ref, bp_ref, out_ref):
    x = emb_ref[...]
    static = jnp.maximum(
        jnp.dot(x, Wst_ref[...], preferred_element_type=jnp.float32)
        + bst_ref[...], 0.0)
    bc = bcol_ref[...]
    br = brow_ref[...]
    ri = lax.broadcasted_iota(jnp.int32, (P, P), 0)
    ci = lax.broadcasted_iota(jnp.int32, (P, P), 1)
    valid = (bc == br) & (ri != ci)
    scale = 1.0 / np.sqrt(float(HD))
    for l in range(2):
      Q = jnp.dot(x, Wq_ref[l], preferred_element_type=jnp.float32) + bq_ref[l]
      K = jnp.dot(x, Wk_ref[l], preferred_element_type=jnp.float32) + bk_ref[l]
      V = jnp.dot(x, Wv_ref[l], preferred_element_type=jnp.float32) + bv_ref[l]
      cols = []
      for h in range(NH):
        Qh = Q[:, h * HD:(h + 1) * HD]
        Kh = K[:, h * HD:(h + 1) * HD]
        Vh = V[:, h * HD:(h + 1) * HD]
        a = lax.dot_general(Qh, Kh, (((1,), (1,)), ((), ())),
                            preferred_element_type=jnp.float32) * scale
        a = jnp.where(valid, a, -1e9)
        m = jnp.max(a, axis=1, keepdims=True)
        ex = jnp.exp(a - m)
        w = ex / jnp.sum(ex, axis=1, keepdims=True)
        cols.append(jnp.dot(w, Vh, preferred_element_type=jnp.float32))
      ctx = jnp.concatenate(cols, axis=1)
      x = x + beta_ref[l, 0] * (
          jnp.dot(ctx, Wo_ref[l], preferred_element_type=jnp.float32)
          + bo_ref[l])
    diff = x - static
    node = jnp.dot(diff * diff, Wp_ref[...],
                   preferred_element_type=jnp.float32) + bp_ref[...]
    colid = lax.broadcasted_iota(jnp.int32, (P, B), 1).astype(jnp.float32)
    onehot = (bc == colid).astype(jnp.float32)
    ssum = lax.dot_general(onehot, node, (((0,), (0,)), ((), ())),
                           preferred_element_type=jnp.float32)
    ones_col = jnp.ones((P, 1), jnp.float32)
    cnt = lax.dot_general(onehot, ones_col, (((0,), (0,)), ((), ())),
                          preferred_element_type=jnp.float32)
    out_ref[...] = ssum / jnp.maximum(cnt, 1.0)

  full = lambda s: pl.BlockSpec(s, lambda: tuple(0 for _ in s))
  return pl.pallas_call(
      body,
      in_specs=[
          full((P, D)), full((P, 1)), full((1, P)),
          full((D, D)), full((1, D)),
          full((2, D, D)), full((2, 1, D)),
          full((2, D, D)), full((2, 1, D)),
          full((2, D, D)), full((2, 1, D)),
          full((2, D, D)), full((2, 1, D)),
          full((2, 1)), full((D, 1)), full((1, 1)),
      ],
      out_specs=full((B, 1)),
      out_shape=jax.ShapeDtypeStruct((B, 1), jnp.float32),
  )(emb, bcol, brow, Wst, bst, Wq, bq, Wk, bk, Wv, bv, Wo, bo, beta, Wp, bp)


def _arrange(v, fill):
  EH = NS * NCH_H * CS
  heavy = v[:EH].reshape(NS, NCH_H, CS)
  light = jnp.concatenate(
      [v[EH:].reshape(NS, NCH_L, CS),
       jnp.full((NS, NCH_H - NCH_L, CS), fill, jnp.int32)], axis=1)
  pair = (light, heavy) if HEAVY_CID == 1 else (heavy, light)
  return jnp.stack(pair, axis=1).reshape(NW, NCH_H, CS)


def _prep_edges(src, dst):
  pad = E_PAD - E
  src = jnp.concatenate([src.astype(jnp.int32), jnp.zeros((pad,), jnp.int32)])
  dst = jnp.concatenate([dst.astype(jnp.int32),
                         jnp.full((pad,), NP - 1, jnp.int32)])
  return _arrange(src, 0), _arrange(dst, NP - 1)


def kernel(edge_index, pert_indices, batch_indices, gene_emb,
           Wl1, bl1, Wr1, Wl2, bl2, Wr2, Wm, bm, vm, Wst, bst,
           Wq, bq, Wk, bk, Wv, bv, Wo, bo, beta, Wp, bp):
  src0, dst0 = _prep_edges(edge_index[0, 0], edge_index[0, 1])
  src1, dst1 = _prep_edges(edge_index[1, 0], edge_index[1, 1])

  part10, deg0 = _segmean_partials(gene_emb, src0, dst0)
  part11, deg1 = _segmean_partials(gene_emb, src1, dst1)
  degT0 = deg0.T
  degT1 = deg1.T

  h10 = _sage_linear(part10, degT0, gene_emb,
                     Wl1[0], bl1[0].reshape(1, D), Wr1[0])
  h11 = _sage_linear(part11, degT1, gene_emb,
                     Wl1[1], bl1[1].reshape(1, D), Wr1[1])

  part20, _ = _segmean_partials(h10, src0, dst0, want_deg=False)
  part21, _ = _segmean_partials(h11, src1, dst1, want_deg=False)

  meta = _layer2_meta(part20, part21, degT0, degT1, h10, h11,
                      Wl2[0], bl2[0].reshape(1, D), Wr2[0],
                      Wl2[1], bl2[1].reshape(1, D), Wr2[1],
                      Wm, bm.reshape(1, D), vm.reshape(D, 1))

  emb = _gather_rows(meta, pert_indices.astype(jnp.int32))

  bf = batch_indices.astype(jnp.float32)
  scores = _hypersagnn(emb, bf.reshape(P, 1), bf.reshape(1, P),
                       Wst, bst.reshape(1, D),
                       Wq, bq.reshape(2, 1, D), Wk, bk.reshape(2, 1, D),
                       Wv, bv.reshape(2, 1, D), Wo, bo.reshape(2, 1, D),
                       beta.reshape(2, 1), Wp, bp.reshape(1, 1))
  return scores[:, 0]


# speedup vs baseline: 1.6351x; 1.5365x over previous
import functools

import jax
import jax.numpy as jnp
import numpy as np
from jax import lax
from jax.experimental import pallas as pl
from jax.experimental.pallas import tpu as pltpu
from jax.experimental.pallas import tpu_sc as plsc

N = 10000
D = 128
E = 320000
T = 2
P = 768
B = 256
NH = 4
HD = D // NH

NC = 2
NS = 16
NW = NC * NS
CS = 128
NCH_H = 79
NCH_L = 79
HEAVY_CID = 0
E_PAD = NS * CS * (NCH_H + NCH_L)
NP = 10240
RPS = NP // NS

_mesh_cache = []


def _mesh():
  if not _mesh_cache:
    _mesh_cache.append(
        plsc.VectorSubcoreMesh(core_axis_name="c", subcore_axis_name="s",
                               num_cores=NC, num_subcores=NS))
  return _mesh_cache[0]


def _segmean_partials(x, src_idx, dst_idx, want_deg=True):
  @functools.partial(
      pl.kernel, mesh=_mesh(),
      out_type=[jax.ShapeDtypeStruct((NC, NP, D), jnp.float32),
                jax.ShapeDtypeStruct((NC, NP), jnp.float32)],
      scratch_types=[
          pltpu.VMEM((NCH_H, CS), jnp.int32),
          pltpu.VMEM((NCH_H, CS), jnp.int32),
          pltpu.VMEM((CS, D), jnp.float32),
          pltpu.VMEM((CS,), jnp.float32),
          pltpu.VMEM_SHARED((NP, D), jnp.float32),
          pltpu.VMEM_SHARED((NP,), jnp.float32),
          pltpu.SemaphoreType.DMA,
      ],
  )
  def k(x_hbm, src_hbm, dst_hbm, part_hbm, deg_hbm,
        src_v, dst_v, r0, ones_v, acc_sh, dacc_sh, s0):
    cid = lax.axis_index("c")
    sid = lax.axis_index("s")
    wid = sid * NC + cid

    zeros16 = jnp.zeros((16,), jnp.float32)
    ones16 = jnp.ones((16,), jnp.float32)

    @pl.loop(0, CS)
    def _(r):
      for c in range(D // 16):
        r0[r, pl.ds(c * 16, 16)] = zeros16
    if want_deg:
      for c in range(CS // 16):
        ones_v[pl.ds(c * 16, 16)] = ones16

    for t in range(RPS // CS):
      pltpu.sync_copy(r0, acc_sh.at[pl.ds(sid * RPS + t * CS, CS)])
    if want_deg:
      for t in range(RPS // D):
        pltpu.sync_copy(r0.at[0],
                        dacc_sh.at[pl.ds(sid * RPS + t * D, D)])
    plsc.subcore_barrier()

    pltpu.sync_copy(src_hbm.at[wid], src_v)
    pltpu.sync_copy(dst_hbm.at[wid], dst_v)

    @pl.loop(0, NCH_H)
    def _(j):
      pltpu.async_copy(x_hbm.at[src_v.at[j]], r0, s0).wait()
      pltpu.sync_copy(r0, acc_sh.at[dst_v.at[j]], add=True)
      if want_deg:
        pltpu.sync_copy(ones_v, dacc_sh.at[dst_v.at[j]], add=True)

    plsc.subcore_barrier()

    pltpu.sync_copy(acc_sh.at[pl.ds(sid * RPS, RPS)],
                    part_hbm.at[cid, pl.ds(sid * RPS, RPS)])
    if want_deg:
      pltpu.sync_copy(dacc_sh.at[pl.ds(sid * RPS, RPS)],
                      deg_hbm.at[cid, pl.ds(sid * RPS, RPS)])

  return k(x, src_idx, dst_idx)


_PPW = P // NW


def _gather_rows(table, idx):
  @functools.partial(
      pl.kernel, mesh=_mesh(),
      out_type=jax.ShapeDtypeStruct((P, D), jnp.float32),
      scratch_types=[
          pltpu.VMEM((_PPW,), jnp.int32),
          pltpu.VMEM((_PPW, D), jnp.float32),
          pltpu.SemaphoreType.DMA,
      ],
  )
  def k(tab_hbm, idx_hbm, out_hbm, idx_v, rows_v, sem):
    wid = lax.axis_index("s") * NC + lax.axis_index("c")
    base = wid * _PPW
    pltpu.sync_copy(idx_hbm.at[pl.ds(base, _PPW)], idx_v)
    pltpu.async_copy(tab_hbm.at[idx_v], rows_v, sem).wait()
    pltpu.sync_copy(rows_v, out_hbm.at[pl.ds(base, _PPW)])

  return k(table, idx)


_BR = 2000


def _sage_linear(part, degT, x, Wl, bl, Wr):
  def body(part_ref, degT_ref, x_ref, Wl_ref, bl_ref, Wr_ref, out_ref):
    p = part_ref[0] + part_ref[1]
    deg = degT_ref[:, 0] + degT_ref[:, 1]
    mean = p / jnp.maximum(deg, 1.0)[:, None]
    h = (jnp.dot(mean, Wl_ref[...], preferred_element_type=jnp.float32)
         + bl_ref[...]
         + jnp.dot(x_ref[...], Wr_ref[...], preferred_element_type=jnp.float32))
    out_ref[...] = jnp.maximum(h, 0.0)

  return pl.pallas_call(
      body,
      grid=(N // _BR,),
      in_specs=[
          pl.BlockSpec((NC, _BR, D), lambda i: (0, i, 0)),
          pl.BlockSpec((_BR, NC), lambda i: (i, 0)),
          pl.BlockSpec((_BR, D), lambda i: (i, 0)),
          pl.BlockSpec((D, D), lambda i: (0, 0)),
          pl.BlockSpec((1, D), lambda i: (0, 0)),
          pl.BlockSpec((D, D), lambda i: (0, 0)),
      ],
      out_specs=pl.BlockSpec((_BR, D), lambda i: (i, 0)),
      out_shape=jax.ShapeDtypeStruct((N, D), jnp.float32),
  )(part, degT, x, Wl, bl, Wr)


def _layer2_meta(part20, part21, degT0, degT1, h0, h1,
                 Wl20, bl20, Wr20, Wl21, bl21, Wr21, Wm, bm, vm):
  def body(p0_ref, p1_ref, d0_ref, d1_ref, h0_ref, h1_ref,
           Wl20_ref, bl20_ref, Wr20_ref, Wl21_ref, bl21_ref, Wr21_ref,
           Wm_ref, bm_ref, vm_ref, out_ref):
    h2 = []
    for (p_ref, d_ref, h_ref, Wl_ref, bl_ref, Wr_ref) in (
        (p0_ref, d0_ref, h0_ref, Wl20_ref, bl20_ref, Wr20_ref),
        (p1_ref, d1_ref, h1_ref, Wl21_ref, bl21_ref, Wr21_ref)):
      p = p_ref[0] + p_ref[1]
      deg = d_ref[:, 0] + d_ref[:, 1]
      mean = p / jnp.maximum(deg, 1.0)[:, None]
      h = (jnp.dot(mean, Wl_ref[...], preferred_element_type=jnp.float32)
           + bl_ref[...]
           + jnp.dot(h_ref[...], Wr_ref[...], preferred_element_type=jnp.float32))
      h2.append(jnp.maximum(h, 0.0))
    Wm_ = Wm_ref[...]
    bm_ = bm_ref[...]
    vm_ = vm_ref[...]
    s0 = jnp.dot(jnp.tanh(jnp.dot(h2[0], Wm_,
                                  preferred_element_type=jnp.float32) + bm_),
                 vm_, preferred_element_type=jnp.float32)
    s1 = jnp.dot(jnp.tanh(jnp.dot(h2[1], Wm_,
                                  preferred_element_type=jnp.float32) + bm_),
                 vm_, preferred_element_type=jnp.float32)
    m = jnp.maximum(s0, s1)
    e0 = jnp.exp(s0 - m)
    e1 = jnp.exp(s1 - m)
    w0 = e0 / (e0 + e1)
    out_ref[...] = w0 * h2[0] + (1.0 - w0) * h2[1]

  wspec = pl.BlockSpec((D, D), lambda i: (0, 0))
  bspec = pl.BlockSpec((1, D), lambda i: (0, 0))
  return pl.pallas_call(
      body,
      grid=(N // _BR,),
      in_specs=[
          pl.BlockSpec((NC, _BR, D), lambda i: (0, i, 0)),
          pl.BlockSpec((NC, _BR, D), lambda i: (0, i, 0)),
          pl.BlockSpec((_BR, NC), lambda i: (i, 0)),
          pl.BlockSpec((_BR, NC), lambda i: (i, 0)),
          pl.BlockSpec((_BR, D), lambda i: (i, 0)),
          pl.BlockSpec((_BR, D), lambda i: (i, 0)),
          wspec, bspec, wspec, wspec, bspec, wspec,
          wspec, bspec, pl.BlockSpec((D, 1), lambda i: (0, 0)),
      ],
      out_specs=pl.BlockSpec((_BR, D), lambda i: (i, 0)),
      out_shape=jax.ShapeDtypeStruct((N, D), jnp.float32),
  )(part20, part21, degT0, degT1, h0, h1,
    Wl20, bl20, Wr20, Wl21, bl21, Wr21, Wm, bm, vm)


def _hypersagnn(emb, bcol, brow, Wst, bst, Wq, bq, Wk, bk, Wv, bv, Wo, bo,
                beta, Wp, bp):
  def body(emb_ref, bcol_ref, brow_ref, Wst_ref, bst_ref,
           Wq_ref, bq_ref, Wk_ref, bk_ref, Wv_ref, bv_ref, Wo_ref, bo_ref,
           beta_ref, Wp_ref, bp_ref, out_ref):
    x = emb_ref[...]
    static = jnp.maximum(
        jnp.dot(x, Wst_ref[...], preferred_element_type=jnp.float32)
        + bst_ref[...], 0.0)
    bc = bcol_ref[...]
    br = brow_ref[...]
    ri = lax.broadcasted_iota(jnp.int32, (P, P), 0)
    ci = lax.broadcasted_iota(jnp.int32, (P, P), 1)
    valid = (bc == br) & (ri != ci)
    scale = 1.0 / np.sqrt(float(HD))
    for l in range(2):
      Q = jnp.dot(x, Wq_ref[l], preferred_element_type=jnp.float32) + bq_ref[l]
      K = jnp.dot(x, Wk_ref[l], preferred_element_type=jnp.float32) + bk_ref[l]
      V = jnp.dot(x, Wv_ref[l], preferred_element_type=jnp.float32) + bv_ref[l]
      cols = []
      for h in range(NH):
        Qh = Q[:, h * HD:(h + 1) * HD]
        Kh = K[:, h * HD:(h + 1) * HD]
        Vh = V[:, h * HD:(h + 1) * HD]
        a = lax.dot_general(Qh, Kh, (((1,), (1,)), ((), ())),
                            preferred_element_type=jnp.float32) * scale
        a = jnp.where(valid, a, -1e9)
        m = jnp.max(a, axis=1, keepdims=True)
        ex = jnp.exp(a - m)
        w = ex / jnp.sum(ex, axis=1, keepdims=True)
        cols.append(jnp.dot(w, Vh, preferred_element_type=jnp.float32))
      ctx = jnp.concatenate(cols, axis=1)
      x = x + beta_ref[l, 0] * (
          jnp.dot(ctx, Wo_ref[l], preferred_element_type=jnp.float32)
          + bo_ref[l])
    diff = x - static
    node = jnp.dot(diff * diff, Wp_ref[...],
                   preferred_element_type=jnp.float32) + bp_ref[...]
    colid = lax.broadcasted_iota(jnp.int32, (P, B), 1).astype(jnp.float32)
    onehot = (bc == colid).astype(jnp.float32)
    ssum = lax.dot_general(onehot, node, (((0,), (0,)), ((), ())),
                           preferred_element_type=jnp.float32)
    ones_col = jnp.ones((P, 1), jnp.float32)
    cnt = lax.dot_general(onehot, ones_col, (((0,), (0,)), ((), ())),
                          preferred_element_type=jnp.float32)
    out_ref[...] = ssum / jnp.maximum(cnt, 1.0)

  full = lambda s: pl.BlockSpec(s, lambda: tuple(0 for _ in s))
  return pl.pallas_call(
      body,
      in_specs=[
          full((P, D)), full((P, 1)), full((1, P)),
          full((D, D)), full((1, D)),
          full((2, D, D)), full((2, 1, D)),
          full((2, D, D)), full((2, 1, D)),
          full((2, D, D)), full((2, 1, D)),
          full((2, D, D)), full((2, 1, D)),
          full((2, 1)), full((D, 1)), full((1, 1)),
      ],
      out_specs=full((B, 1)),
      out_shape=jax.ShapeDtypeStruct((B, 1), jnp.float32),
  )(emb, bcol, brow, Wst, bst, Wq, bq, Wk, bk, Wv, bv, Wo, bo, beta, Wp, bp)


def _arrange(v, fill):
  EH = NS * NCH_H * CS
  heavy = v[:EH].reshape(NS, NCH_H, CS)
  light = jnp.concatenate(
      [v[EH:].reshape(NS, NCH_L, CS),
       jnp.full((NS, NCH_H - NCH_L, CS), fill, jnp.int32)], axis=1)
  pair = (light, heavy) if HEAVY_CID == 1 else (heavy, light)
  return jnp.stack(pair, axis=1).reshape(NW, NCH_H, CS)


def _prep_edges(src, dst):
  pad = E_PAD - E
  src = jnp.concatenate([src.astype(jnp.int32), jnp.zeros((pad,), jnp.int32)])
  dst = jnp.concatenate([dst.astype(jnp.int32),
                         jnp.full((pad,), NP - 1, jnp.int32)])
  return _arrange(src, 0), _arrange(dst, NP - 1)


def kernel(edge_index, pert_indices, batch_indices, gene_emb,
           Wl1, bl1, Wr1, Wl2, bl2, Wr2, Wm, bm, vm, Wst, bst,
           Wq, bq, Wk, bk, Wv, bv, Wo, bo, beta, Wp, bp):
  src0, dst0 = _prep_edges(edge_index[0, 0], edge_index[0, 1])
  src1, dst1 = _prep_edges(edge_index[1, 0], edge_index[1, 1])

  part10, deg0 = _segmean_partials(gene_emb, src0, dst0)
  part11, deg1 = _segmean_partials(gene_emb, src1, dst1)
  degT0 = deg0.T
  degT1 = deg1.T

  h10 = _sage_linear(part10, degT0, gene_emb,
                     Wl1[0], bl1[0].reshape(1, D), Wr1[0])
  h11 = _sage_linear(part11, degT1, gene_emb,
                     Wl1[1], bl1[1].reshape(1, D), Wr1[1])

  part20, _ = _segmean_partials(h10, src0, dst0, want_deg=False)
  part21, _ = _segmean_partials(h11, src1, dst1, want_deg=False)

  meta = _layer2_meta(part20, part21, degT0, degT1, h10, h11,
                      Wl2[0], bl2[0].reshape(1, D), Wr2[0],
                      Wl2[1], bl2[1].reshape(1, D), Wr2[1],
                      Wm, bm.reshape(1, D), vm.reshape(D, 1))

  emb = _gather_rows(meta, pert_indices.astype(jnp.int32))

  bf = batch_indices.astype(jnp.float32)
  scores = _hypersagnn(emb, bf.reshape(P, 1), bf.reshape(1, P),
                       Wst, bst.reshape(1, D),
                       Wq, bq.reshape(2, 1, D), Wk, bk.reshape(2, 1, D),
                       Wv, bv.reshape(2, 1, D), Wo, bo.reshape(2, 1, D),
                       beta.reshape(2, 1), Wp, bp.reshape(1, 1))
  return scores[:, 0]
